# baseline (speedup 1.0000x reference)
import functools

import jax
import jax.numpy as jnp
import numpy as np
from jax import lax
from jax.experimental import pallas as pl
from jax.experimental.pallas import tpu as pltpu
from jax.experimental.pallas import tpu_sc as plsc

B = 2
C = 80
NPRIME = 506880
PPB = NPRIME // B
NS = 16
PPT = PPB // NS
CH = 16
NPASS = C // CH
GRID = 200
CELLS = GRID * GRID
DUMP = CELLS
ACC_ROWS = CELLS + 8
CHUNK = 96
NCHUNK = PPT // CHUNK
CPS = 5
SLAB = CHUNK * CPS
NBUF = 3
NGRP = PPT // (SLAB * NBUF)
ROWS_PER_TILE = CELLS // NS
ZROWS = 250

_DX = np.float32(0.005)
_DZ = np.float32(1.0)

_mesh = plsc.VectorSubcoreMesh(core_axis_name="c", subcore_axis_name="s")


@functools.partial(
    pl.kernel,
    mesh=_mesh,
    compiler_params=pltpu.CompilerParams(use_tc_tiling_on_sc=False),
    out_type=jax.ShapeDtypeStruct((B, NPASS, CELLS, CH), jnp.float32),
    scratch_types=[
        pltpu.VMEM((3, PPT), jnp.float32),
        pltpu.VMEM((NCHUNK, CHUNK), jnp.int32),
        pltpu.VMEM((NBUF, SLAB, CH), jnp.float32),
        pltpu.VMEM((ZROWS, CH), jnp.float32),
        pltpu.VMEM_SHARED((ACC_ROWS, CH), jnp.float32),
        pltpu.SemaphoreType.DMA,
        pltpu.SemaphoreType.DMA,
        pltpu.SemaphoreType.DMA,
        pltpu.SemaphoreType.DMA,
    ],
)
def _bevpool_sc(gx_hbm, gy_hbm, gz_hbm, x_hbm, out_hbm, gbuf, idxbuf, xbuf,
                zbuf, acc, lsem0, lsem1, lsem2, ssem):
    lsems = (lsem0, lsem1, lsem2)
    c = lax.axis_index("c")
    s = lax.axis_index("s")
    gbase = c * PPB + s * PPT
    zvec = jnp.zeros((16,), jnp.float32)

    def _fill_zero(i, carry):
        zbuf[i, :] = zvec
        return carry

    lax.fori_loop(0, ZROWS, _fill_zero, 0)

    for comp, comp_hbm in enumerate((gx_hbm, gy_hbm, gz_hbm)):
        pltpu.sync_copy(comp_hbm.at[pl.ds(gbase, PPT)], gbuf.at[comp])

    def _voxelize(r, carry):
        for cc in range(CHUNK // 16):
            o = r * CHUNK + cc * 16
            vx = gbuf[0, pl.ds(o, 16)]
            vy = gbuf[1, pl.ds(o, 16)]
            vz = gbuf[2, pl.ds(o, 16)]
            ix = (vx / _DX).astype(jnp.int32)
            iy = (vy / _DX).astype(jnp.int32)
            iz = (vz / _DZ).astype(jnp.int32)
            kept = (
                (ix >= 0) & (ix < GRID)
                & (iy >= 0) & (iy < GRID)
                & (iz >= 0) & (iz < 1)
            )
            lin = ix * GRID + iy
            idxbuf[r, pl.ds(cc * 16, 16)] = jnp.where(kept, lin, DUMP)
        return carry

    lax.fori_loop(0, NCHUNK, _voxelize, 0)

    for p in range(NPASS):
        for z in range(ROWS_PER_TILE // ZROWS):
            pltpu.sync_copy(
                zbuf, acc.at[pl.ds(s * ROWS_PER_TILE + z * ZROWS, ZROWS)])
        plsc.subcore_barrier()

        def _group(m, carry, p=p):
            t0 = m * NBUF
            loads = []
            for k in range(NBUF):
                loads.append(pltpu.async_copy(
                    x_hbm.at[pl.ds(gbase + (t0 + k) * SLAB, SLAB),
                             pl.ds(p * CH, CH)],
                    xbuf.at[k], lsems[k]))
            scats = []
            for k in range(NBUF):
                loads[k].wait()
                for j in range(CPS):
                    scats.append(pltpu.async_copy(
                        xbuf.at[k, pl.ds(j * CHUNK, CHUNK)],
                        acc.at[idxbuf.at[(t0 + k) * CPS + j]],
                        ssem, add=True))
            for h in scats:
                h.wait()
            return carry

        lax.fori_loop(0, NGRP, _group, 0)
        plsc.subcore_barrier()

        pltpu.sync_copy(
            acc.at[pl.ds(s * ROWS_PER_TILE, ROWS_PER_TILE)],
            out_hbm.at[c, p, pl.ds(s * ROWS_PER_TILE, ROWS_PER_TILE)],
        )
        plsc.subcore_barrier()


_TBLK = 2560


def _tc_t_body(in_ref, out_ref):
    out_ref[...] = jnp.swapaxes(in_ref[...], -1, -2)


def _tc_transpose(xin):
    bp = B * NPASS
    return pl.pallas_call(
        _tc_t_body,
        grid=(bp, pl.cdiv(CELLS, _TBLK)),
        in_specs=[pl.BlockSpec((1, _TBLK, CH), lambda i, j: (i, j, 0))],
        out_specs=pl.BlockSpec((1, CH, _TBLK), lambda i, j: (i, 0, j)),
        out_shape=jax.ShapeDtypeStruct((bp, CH, CELLS), jnp.float32),
    )(xin)


def kernel(geom_feats, x):
    g = geom_feats.reshape(NPRIME, 3)
    x2d = x.reshape(NPRIME, C)
    out = _bevpool_sc(g[:, 0], g[:, 1], g[:, 2], x2d)
    outt = _tc_transpose(out.reshape(B * NPASS, CELLS, CH))
    return outt.reshape(B, C, GRID, GRID)


# speedup vs baseline: 1.2364x; 1.2364x over previous
import functools

import jax
import jax.numpy as jnp
import numpy as np
from jax import lax
from jax.experimental import pallas as pl
from jax.experimental.pallas import tpu as pltpu
from jax.experimental.pallas import tpu_sc as plsc

B = 2
C = 80
NPRIME = 506880
PPB = NPRIME // B
NS = 16
PPT = PPB // NS
CH = 16
NPASS = C // CH
GRID = 200
CELLS = GRID * GRID
DUMP = CELLS
ACC_ROWS = CELLS + 8
CHUNK = 96
NCHUNK = PPT // CHUNK
CPS = 5
SLAB = CHUNK * CPS
NBUF = 3
NGRP = PPT // (SLAB * NBUF)
ROWS_PER_TILE = CELLS // NS
ZROWS = 250

_mesh = plsc.VectorSubcoreMesh(core_axis_name="c", subcore_axis_name="s")


@functools.partial(
    pl.kernel,
    mesh=_mesh,
    compiler_params=pltpu.CompilerParams(use_tc_tiling_on_sc=False),
    out_type=jax.ShapeDtypeStruct((B, CELLS, C), jnp.float32),
    scratch_types=[
        pltpu.VMEM((NCHUNK, CHUNK), jnp.int32),
        pltpu.VMEM((NBUF, SLAB, CH), jnp.float32),
        pltpu.VMEM((ZROWS, CH), jnp.float32),
        pltpu.VMEM_SHARED((ACC_ROWS, CH), jnp.float32),
        pltpu.SemaphoreType.DMA,
        pltpu.SemaphoreType.DMA,
        pltpu.SemaphoreType.DMA,
        pltpu.SemaphoreType.DMA,
    ],
)
def _bevpool_sc(idx_hbm, x_hbm, out_hbm, idxbuf, xbuf,
                zbuf, acc, lsem0, lsem1, lsem2, ssem):
    lsems = (lsem0, lsem1, lsem2)
    c = lax.axis_index("c")
    s = lax.axis_index("s")
    wid = c * NS + s
    gbase = c * PPB + s * PPT
    zvec = jnp.zeros((16,), jnp.float32)

    def _fill_zero(i, carry):
        zbuf[i, :] = zvec
        return carry

    lax.fori_loop(0, ZROWS, _fill_zero, 0)

    pltpu.sync_copy(idx_hbm.at[wid], idxbuf)

    for p in range(NPASS):
        for z in range(ROWS_PER_TILE // ZROWS):
            pltpu.sync_copy(
                zbuf, acc.at[pl.ds(s * ROWS_PER_TILE + z * ZROWS, ZROWS)])
        plsc.subcore_barrier()

        def _group(m, carry, p=p):
            t0 = m * NBUF
            loads = []
            for k in range(NBUF):
                loads.append(pltpu.async_copy(
                    x_hbm.at[pl.ds(gbase + (t0 + k) * SLAB, SLAB),
                             pl.ds(p * CH, CH)],
                    xbuf.at[k], lsems[k]))
            scats = []
            for k in range(NBUF):
                loads[k].wait()
                for j in range(CPS):
                    scats.append(pltpu.async_copy(
                        xbuf.at[k, pl.ds(j * CHUNK, CHUNK)],
                        acc.at[idxbuf.at[(t0 + k) * CPS + j]],
                        ssem, add=True))
            for h in scats:
                h.wait()
            return carry

        lax.fori_loop(0, NGRP, _group, 0)
        plsc.subcore_barrier()

        pltpu.sync_copy(
            acc.at[pl.ds(s * ROWS_PER_TILE, ROWS_PER_TILE)],
            out_hbm.at[c, pl.ds(s * ROWS_PER_TILE, ROWS_PER_TILE),
                       pl.ds(p * CH, CH)],
        )
        plsc.subcore_barrier()


def kernel(geom_feats, x):
    dxv = jnp.array([0.005, 0.005, 1.0], dtype=jnp.float32)
    bxv = jnp.array([0.0 + 0.005 / 2.0, 0.0 + 0.005 / 2.0, 0.0 + 1.0 / 2.0],
                    dtype=jnp.float32)
    g3 = geom_feats.reshape(NPRIME, 3)
    gf = jnp.trunc((g3 - (bxv - dxv / 2.0)) / dxv).astype(jnp.int32)
    ix, iy, iz = gf[:, 0], gf[:, 1], gf[:, 2]
    kept = ((ix >= 0) & (ix < GRID) & (iy >= 0) & (iy < GRID)
            & (iz >= 0) & (iz < 1))
    idx = jnp.where(kept, ix * GRID + iy, DUMP).astype(jnp.int32)

    x2d = x.reshape(NPRIME, C)
    out = _bevpool_sc(idx.reshape(B * NS, NCHUNK, CHUNK), x2d)
    return out.transpose(0, 2, 1).reshape(B, C, GRID, GRID)
